# baseline (speedup 1.0000x reference)
import jax
import jax.numpy as jnp
from jax import lax
from jax.experimental import pallas as pl
from jax.experimental.pallas import tpu as pltpu
from jax.experimental.pallas import tpu_sc as plsc

_B = 4096 * 200
_D = 30
_DP = 32
_NC, _NS = 2, 16
_NW = _NC * _NS
_BPW = _B // _NW
_CH = 512
_NCHUNK = _BPW // _CH
_UNROLL = 8


def _emb_body(x_hbm, table_hbm, out_hbm, idx_all,
              rows0, rows1, comp0, comp1, gsem0, gsem1, wsem0, wsem1):
    wid = lax.axis_index("s") * _NC + lax.axis_index("c")
    base = wid * _BPW
    rows = (rows0, rows1)
    comp = (comp0, comp1)
    gsem = (gsem0, gsem1)
    wsem = (wsem0, wsem1)

    pltpu.sync_copy(x_hbm.at[pl.ds(base, _BPW)], idx_all)

    def g_start(c, b):
        pltpu.async_copy(
            table_hbm.at[idx_all.at[pl.ds(c * _CH, _CH)]], rows[b], gsem[b])

    def g_wait(b):
        pltpu.make_async_copy(
            table_hbm.at[idx_all.at[pl.ds(0, _CH)]], rows[b], gsem[b]).wait()

    def w_start(c, b):
        pltpu.async_copy(
            comp[b], out_hbm.at[pl.ds((base + c * _CH) * _D, _CH * _D)],
            wsem[b])

    def w_wait(b):
        pltpu.make_async_copy(
            comp[b], out_hbm.at[pl.ds(0, _CH * _D)], wsem[b]).wait()

    g_start(0, 0)

    def step(c, b):
        @pl.when(c >= 2)
        def _():
            w_wait(b)

        g_wait(b)

        @pl.when(c + 1 < _NCHUNK)
        def _():
            g_start(c + 1, 1 - b)

        def compact(u, carry2):
            r0 = u * _UNROLL
            for k in range(_UNROLL):
                r = r0 + k
                a = rows[b][r, pl.ds(0, 16)]
                t = rows[b][r, pl.ds(14, 16)]
                comp[b][pl.ds(r * _D, 16)] = a
                comp[b][pl.ds(r * _D + 14, 16)] = t
            return carry2

        lax.fori_loop(0, _CH // _UNROLL, compact, 0)

        w_start(c, b)

    def body(p, carry):
        for b in range(2):
            step(2 * p + b, b)
        return carry

    lax.fori_loop(0, _NCHUNK // 2, body, 0)

    w_wait(0)
    w_wait(1)


def kernel(x, table):
    xf = x.reshape(-1)
    tpad = jnp.pad(table, ((0, 0), (0, _DP - _D)))
    mesh = plsc.VectorSubcoreMesh(core_axis_name="c", subcore_axis_name="s")
    f = pl.kernel(
        _emb_body,
        mesh=mesh,
        out_type=jax.ShapeDtypeStruct((_B * _D,), jnp.float32),
        scratch_types=[
            pltpu.VMEM((_BPW,), jnp.int32),
            pltpu.VMEM((_CH, _DP), jnp.float32),
            pltpu.VMEM((_CH, _DP), jnp.float32),
            pltpu.VMEM((_CH * _D,), jnp.float32),
            pltpu.VMEM((_CH * _D,), jnp.float32),
            pltpu.SemaphoreType.DMA,
            pltpu.SemaphoreType.DMA,
            pltpu.SemaphoreType.DMA,
            pltpu.SemaphoreType.DMA,
        ],
        compiler_params=pltpu.CompilerParams(use_tc_tiling_on_sc=False),
    )
    out = f(xf, tpad)
    return out.reshape(x.shape[0], x.shape[1], _D)


# speedup vs baseline: 1.2764x; 1.2764x over previous
import jax
import jax.numpy as jnp
from jax import lax
from jax.experimental import pallas as pl
from jax.experimental.pallas import tpu as pltpu
from jax.experimental.pallas import tpu_sc as plsc

_NI, _NJ = 4096, 200
_B = _NI * _NJ
_D = 30
_DP = 32
_NC, _NS = 2, 16
_NW = _NC * _NS
_BPW = _B // _NW
_TJ = _NJ // 8
_WG = 512
_NWG = _BPW // _WG


def _emb_body(x_hbm, table_hbm, out_hbm, idx_all,
              wgi0, wgi1, rows0, rows1, comp0, comp1,
              gsem0, gsem1, wsem0, wsem1):
    wid = lax.axis_index("s") * _NC + lax.axis_index("c")
    base = wid * _BPW
    wgi = (wgi0, wgi1)
    rows = (rows0, rows1)
    comp = (comp0, comp1)
    gsem = (gsem0, gsem1)
    wsem = (wsem0, wsem1)

    iota = lax.iota(jnp.int32, 16)
    i200 = iota * 200

    pltpu.sync_copy(x_hbm.at[pl.ds(base, _BPW)], idx_all)

    def build_idx(c, b):
        j0 = (c // 2) * 8 + (c % 2) * 4
        for v in range(32):
            jj = v // 8
            r = v % 8
            src = i200 + (3200 * r + j0 + jj)
            vals = plsc.load_gather(idx_all, [src])
            wgi[b][pl.ds(16 * v, 16)] = vals

    def g_start(b):
        pltpu.async_copy(table_hbm.at[wgi[b]], rows[b], gsem[b])

    def g_wait(b):
        pltpu.make_async_copy(table_hbm.at[wgi[b]], rows[b], gsem[b]).wait()

    def produce(b):
        def perk(k, carry):
            for v in range(32):
                rvec = 16 * v + iota
                cvec = jnp.full((16,), k, jnp.int32)
                vals = plsc.load_gather(rows[b], [rvec, cvec])
                comp[b][k, v // 8, pl.ds(16 * (v % 8), 16)] = vals
            return carry

        lax.fori_loop(0, _D, perk, 0)

    def w_start(c, b):
        tj = c // 2
        js = c % 2
        pltpu.async_copy(
            comp[b],
            out_hbm.at[:, tj, wid, pl.ds(4 * js, 4), :],
            wsem[b])

    def w_wait(b):
        pltpu.make_async_copy(
            comp[b], out_hbm.at[:, 0, 0, pl.ds(0, 4), :], wsem[b]).wait()

    build_idx(0, 0)
    g_start(0)

    def step(c, b):
        g_wait(b)

        @pl.when(c + 1 < _NWG)
        def _():
            build_idx(c + 1, 1 - b)
            g_start(1 - b)

        @pl.when(c >= 2)
        def _():
            w_wait(b)

        produce(b)
        w_start(c, b)

    def body(p, carry):
        for b in range(2):
            step(2 * p + b, b)
        return carry

    lax.fori_loop(0, _NWG // 2, body, 0)

    w_wait(0)
    w_wait(1)


def kernel(x, table):
    xf = x.reshape(-1)
    tpad = jnp.pad(table, ((0, 0), (0, _DP - _D)))
    mesh = plsc.VectorSubcoreMesh(core_axis_name="c", subcore_axis_name="s")
    f = pl.kernel(
        _emb_body,
        mesh=mesh,
        out_type=jax.ShapeDtypeStruct((_D, _TJ, _NW, 8, 128), jnp.float32),
        scratch_types=[
            pltpu.VMEM((_BPW,), jnp.int32),
            pltpu.VMEM((_WG,), jnp.int32),
            pltpu.VMEM((_WG,), jnp.int32),
            pltpu.VMEM((_WG, _DP), jnp.float32),
            pltpu.VMEM((_WG, _DP), jnp.float32),
            pltpu.VMEM((_D, 4, 128), jnp.float32),
            pltpu.VMEM((_D, 4, 128), jnp.float32),
            pltpu.SemaphoreType.DMA,
            pltpu.SemaphoreType.DMA,
            pltpu.SemaphoreType.DMA,
            pltpu.SemaphoreType.DMA,
        ],
        compiler_params=pltpu.CompilerParams(
            use_tc_tiling_on_sc=False, needs_layout_passes=False),
    )
    out5 = f(xf, tpad)
    return out5.transpose(2, 4, 1, 3, 0).reshape(_NI, _NJ, _D)


# speedup vs baseline: 2.7454x; 2.1509x over previous
import jax
import jax.numpy as jnp
from jax import lax
from jax.experimental import pallas as pl
from jax.experimental.pallas import tpu as pltpu
from jax.experimental.pallas import tpu_sc as plsc

_NI, _NJ = 4096, 200
_B = _NI * _NJ
_D = 30
_DP = 32
_NC, _NS = 2, 16
_NW = _NC * _NS
_BPW = _B // _NW
_TJ = _NJ // 8
_WG = 512
_NWG = _BPW // _WG


def _emb_body(x_hbm, table_hbm, out_hbm, idx_all,
              wgi0, wgi1, rows0, rows1, comp0, comp1,
              gsem0, gsem1, wsem0, wsem1):
    wid = lax.axis_index("s") * _NC + lax.axis_index("c")
    base = wid * _BPW
    wgi = (wgi0, wgi1)
    rows = (rows0, rows1)
    comp = (comp0, comp1)
    gsem = (gsem0, gsem1)
    wsem = (wsem0, wsem1)

    iota = lax.iota(jnp.int32, 16)
    i200 = iota * 200

    pltpu.sync_copy(x_hbm.at[pl.ds(base, _BPW)], idx_all)

    def build_idx(c, b):
        j0 = (c // 2) * 8 + (c % 2) * 4
        for v in range(32):
            jj = v // 8
            r = v % 8
            src = i200 + (3200 * r + j0 + jj)
            vals = plsc.load_gather(idx_all, [src])
            wgi[b][pl.ds(16 * v, 16)] = vals

    def g_start(b):
        pltpu.async_copy(table_hbm.at[wgi[b]], rows[b], gsem[b])

    def g_wait(b):
        pltpu.make_async_copy(table_hbm.at[wgi[b]], rows[b], gsem[b]).wait()

    pds = [(iota + d) & 15 for d in range(16)]

    def produce(b):
        for h in (0, 1):
            kbase = 14 * h

            def blk(rb, carry):
                r0 = 16 * rb
                rvec = r0 + iota
                jjv = jnp.full((16,), rb // 8, jnp.int32)
                iiv = 16 * (rb % 8) + iota
                for d in range(16):
                    kv = pds[d] + kbase
                    vals = plsc.load_gather(rows[b], [rvec, kv])
                    plsc.store_scatter(comp[b], [kv, jjv, iiv], vals)
                return carry

            lax.fori_loop(0, _WG // 16, blk, 0)

    def w_start(c, b):
        tj = c // 2
        js = c % 2
        pltpu.async_copy(
            comp[b],
            out_hbm.at[:, tj, wid, pl.ds(4 * js, 4), :],
            wsem[b])

    def w_wait(b):
        pltpu.make_async_copy(
            comp[b], out_hbm.at[:, 0, 0, pl.ds(0, 4), :], wsem[b]).wait()

    build_idx(0, 0)
    g_start(0)

    def step(c, b):
        g_wait(b)

        @pl.when(c + 1 < _NWG)
        def _():
            build_idx(c + 1, 1 - b)
            g_start(1 - b)

        @pl.when(c >= 2)
        def _():
            w_wait(b)

        produce(b)
        w_start(c, b)

    def body(p, carry):
        for b in range(2):
            step(2 * p + b, b)
        return carry

    lax.fori_loop(0, _NWG // 2, body, 0)

    w_wait(0)
    w_wait(1)


def kernel(x, table):
    xf = x.reshape(-1)
    tpad = jnp.pad(table, ((0, 0), (0, _DP - _D)))
    mesh = plsc.VectorSubcoreMesh(core_axis_name="c", subcore_axis_name="s")
    f = pl.kernel(
        _emb_body,
        mesh=mesh,
        out_type=jax.ShapeDtypeStruct((_D, _TJ, _NW, 8, 128), jnp.float32),
        scratch_types=[
            pltpu.VMEM((_BPW,), jnp.int32),
            pltpu.VMEM((_WG,), jnp.int32),
            pltpu.VMEM((_WG,), jnp.int32),
            pltpu.VMEM((_WG, _DP), jnp.float32),
            pltpu.VMEM((_WG, _DP), jnp.float32),
            pltpu.VMEM((_D, 4, 128), jnp.float32),
            pltpu.VMEM((_D, 4, 128), jnp.float32),
            pltpu.SemaphoreType.DMA,
            pltpu.SemaphoreType.DMA,
            pltpu.SemaphoreType.DMA,
            pltpu.SemaphoreType.DMA,
        ],
        compiler_params=pltpu.CompilerParams(
            use_tc_tiling_on_sc=False, needs_layout_passes=False),
    )
    out5 = f(xf, tpad)
    return out5.transpose(2, 4, 1, 3, 0).reshape(_NI, _NJ, _D)


# speedup vs baseline: 4.5209x; 1.6468x over previous
import jax
import jax.numpy as jnp
from jax import lax
from jax.experimental import pallas as pl
from jax.experimental.pallas import tpu as pltpu
from jax.experimental.pallas import tpu_sc as plsc

_NI, _NJ = 4096, 200
_B = _NI * _NJ
_D = 30
_DP = 32
_NC, _NS = 2, 16
_NW = _NC * _NS
_BPW = _B // _NW
_TJ = _NJ // 8
_WG = 512
_NWG = _BPW // _WG


def _emb_body(x_hbm, table_hbm, out_hbm, idx_all,
              wgi0, wgi1, rows0, rows1, comp0, comp1,
              gsem0, gsem1, wsem0, wsem1):
    wid = lax.axis_index("s") * _NC + lax.axis_index("c")
    base = wid * _BPW
    wgi = (wgi0, wgi1)
    rows = (rows0, rows1)
    comp = (comp0, comp1)
    gsem = (gsem0, gsem1)
    wsem = (wsem0, wsem1)

    iota = lax.iota(jnp.int32, 16)
    i200 = iota * 200

    pltpu.sync_copy(x_hbm.at[pl.ds(base, _BPW)], idx_all)

    def build_idx(c, b):
        j0 = (c // 2) * 8 + (c % 2) * 4
        for v in range(32):
            jj = v // 8
            r = v % 8
            src = i200 + (3200 * r + j0 + jj)
            vals = plsc.load_gather(idx_all, [src])
            wgi[b][pl.ds(16 * v, 16)] = vals

    def g_start(b):
        pltpu.async_copy(table_hbm.at[wgi[b]], rows[b], gsem[b])

    def g_wait(b):
        pltpu.make_async_copy(table_hbm.at[wgi[b]], rows[b], gsem[b]).wait()

    pds = [(iota + d) & 15 for d in range(16)]

    def produce(b):
        for h in (0, 1):
            kbase = 14 * h

            @plsc.parallel_loop(0, _WG // 16, step=1)
            def blk(rb):
                r0 = 16 * rb
                rvec = r0 + iota
                jjv = jnp.full((16,), rb // 8, jnp.int32)
                iiv = 16 * (rb % 8) + iota
                for d in range(16):
                    kv = pds[d] + kbase
                    vals = plsc.load_gather(rows[b], [rvec, kv])
                    plsc.store_scatter(comp[b], [kv, jjv, iiv], vals)

    def w_start(c, b):
        tj = c // 2
        js = c % 2
        pltpu.async_copy(
            comp[b],
            out_hbm.at[:, tj, wid, pl.ds(4 * js, 4), :],
            wsem[b])

    def w_wait(b):
        pltpu.make_async_copy(
            comp[b], out_hbm.at[:, 0, 0, pl.ds(0, 4), :], wsem[b]).wait()

    build_idx(0, 0)
    g_start(0)

    def step(c, b):
        g_wait(b)

        @pl.when(c + 1 < _NWG)
        def _():
            build_idx(c + 1, 1 - b)
            g_start(1 - b)

        @pl.when(c >= 2)
        def _():
            w_wait(b)

        produce(b)
        w_start(c, b)

    def body(p, carry):
        for b in range(2):
            step(2 * p + b, b)
        return carry

    lax.fori_loop(0, _NWG // 2, body, 0)

    w_wait(0)
    w_wait(1)


def kernel(x, table):
    xf = x.reshape(-1)
    tpad = jnp.pad(table, ((0, 0), (0, _DP - _D)))
    mesh = plsc.VectorSubcoreMesh(core_axis_name="c", subcore_axis_name="s")
    f = pl.kernel(
        _emb_body,
        mesh=mesh,
        out_type=jax.ShapeDtypeStruct((_D, _TJ, _NW, 8, 128), jnp.float32),
        scratch_types=[
            pltpu.VMEM((_BPW,), jnp.int32),
            pltpu.VMEM((_WG,), jnp.int32),
            pltpu.VMEM((_WG,), jnp.int32),
            pltpu.VMEM((_WG, _DP), jnp.float32),
            pltpu.VMEM((_WG, _DP), jnp.float32),
            pltpu.VMEM((_D, 4, 128), jnp.float32),
            pltpu.VMEM((_D, 4, 128), jnp.float32),
            pltpu.SemaphoreType.DMA,
            pltpu.SemaphoreType.DMA,
            pltpu.SemaphoreType.DMA,
            pltpu.SemaphoreType.DMA,
        ],
        compiler_params=pltpu.CompilerParams(
            use_tc_tiling_on_sc=False, needs_layout_passes=False),
    )
    out5 = f(xf, tpad)
    return out5.transpose(2, 4, 1, 3, 0).reshape(_NI, _NJ, _D)
